# baseline (speedup 1.0000x reference)
import functools

import jax
import jax.numpy as jnp
from jax import lax
from jax.experimental import pallas as pl
from jax.experimental.pallas import tpu as pltpu
from jax.experimental.pallas import tpu_sc as plsc

_LANE = 128
_NBUF = 4
_NW = 32


def _build_gather(n_rows, d, v, n_t):
    rows_per_w = n_rows // _NW
    n_iters = rows_per_w // _NBUF
    d_tiles = d // 8
    blk_words = d_tiles * 8 * _LANE

    mesh = plsc.VectorSubcoreMesh(core_axis_name="c", subcore_axis_name="s")

    @functools.partial(
        pl.kernel,
        mesh=mesh,
        out_type=jax.ShapeDtypeStruct((n_t * _LANE * blk_words,),
                                      jnp.float32),
        compiler_params=pltpu.CompilerParams(use_tc_tiling_on_sc=False,
                                             needs_layout_passes=False),
        scratch_types=[
            pltpu.VMEM((_NBUF, 1, _LANE), jnp.int32),
            pltpu.VMEM((_NBUF, _LANE, d), jnp.float32),
            pltpu.VMEM((_NBUF, blk_words), jnp.float32),
            pltpu.SemaphoreType.DMA((_NBUF,)),
            pltpu.SemaphoreType.DMA((_NBUF,)),
            pltpu.SemaphoreType.DMA((_NBUF,)),
        ],
    )
    def gather(idx_hbm, table_hbm, out_hbm, idx_v, rows_v, trans_v,
               sem_i, sem_g, sem_s):
        wid = lax.axis_index("s") * 2 + lax.axis_index("c")
        base = wid * rows_per_w

        iota = lax.iota(jnp.int32, 16)
        patterns = []
        for h in range(d // 16):
            dd = iota + 16 * h
            patterns.append((dd // 8) * (8 * _LANE) + (dd % 8) * _LANE)

        def fire_gather(j):
            pltpu.async_copy(table_hbm.at[idx_v.at[j, 0]], rows_v.at[j],
                             sem_g.at[j])

        def drain_gather(j):
            pltpu.make_async_copy(
                table_hbm.at[pl.ds(0, _LANE)], rows_v.at[j],
                sem_g.at[j]).wait()

        def fire_idx(j, k):
            pltpu.async_copy(idx_hbm.at[pl.ds(k, 1)], idx_v.at[j],
                             sem_i.at[j])

        def drain_idx(j):
            pltpu.make_async_copy(
                idx_hbm.at[pl.ds(0, 1)], idx_v.at[j], sem_i.at[j]).wait()

        def fire_store(j, k):
            t = k // _LANE
            c = lax.rem(k, _LANE)
            for r in range(d_tiles):
                off = (t * d_tiles * _LANE + r * _LANE + c) * 8 * _LANE
                pltpu.async_copy(
                    trans_v.at[j, pl.ds(r * 8 * _LANE, 8 * _LANE)],
                    out_hbm.at[pl.ds(off, 8 * _LANE)], sem_s.at[j])

        def drain_store(j):
            pltpu.make_async_copy(
                trans_v.at[j], out_hbm.at[pl.ds(0, blk_words)],
                sem_s.at[j]).wait()

        def transpose_block(j):
            @plsc.parallel_loop(0, _LANE, 1, unroll=16)
            def _t(l):
                for h in range(d // 16):
                    vals = rows_v[j, l, pl.ds(16 * h, 16)]
                    plsc.store_scatter(trans_v.at[j], [patterns[h] + l],
                                       vals)

        for j in range(_NBUF):
            pltpu.sync_copy(idx_hbm.at[pl.ds(base + j, 1)], idx_v.at[j])
            fire_gather(j)

        def body(r, carry):
            for j in range(_NBUF):
                k = base + r * _NBUF + j
                drain_gather(j)

                @pl.when(r < n_iters - 1)
                def _prefetch_idx():
                    fire_idx(j, k + _NBUF)

                @pl.when(r > 0)
                def _wait_prev_store():
                    drain_store(j)

                transpose_block(j)
                fire_store(j, k)

                @pl.when(r < n_iters - 1)
                def _refill():
                    drain_idx(j)
                    fire_gather(j)
            return carry

        lax.fori_loop(0, n_iters, body, 0, unroll=False)

        for j in range(_NBUF):
            drain_store(j)

    return gather


def kernel(x, emb):
    b0, b1 = x.shape
    v, d = emb.shape
    n = b0 * b1
    n_rows = n // _LANE
    idx = x.T.reshape(n_rows, _LANE).astype(jnp.int32)
    out = _build_gather(n_rows, d, v, b1)(idx, emb)
    out = out.reshape(b1, d // 8, _LANE, 8, _LANE)
    return out.transpose(2, 4, 0, 1, 3).reshape(b0, b1, d)


# speedup vs baseline: 1.9108x; 1.9108x over previous
import functools

import jax
import jax.numpy as jnp
from jax import lax
from jax.experimental import pallas as pl
from jax.experimental.pallas import tpu as pltpu
from jax.experimental.pallas import tpu_sc as plsc

_LANE = 128
_NBUF = 4
_NW = 32


def _build_gather(n_rows, d, v, n_t):
    rows_per_w = n_rows // _NW
    n_iters = rows_per_w // _NBUF
    d_tiles = d // 8
    blk_words = d_tiles * 8 * _LANE

    mesh = plsc.VectorSubcoreMesh(core_axis_name="c", subcore_axis_name="s")

    @functools.partial(
        pl.kernel,
        mesh=mesh,
        out_type=jax.ShapeDtypeStruct((n_t * _LANE * blk_words,),
                                      jnp.float32),
        compiler_params=pltpu.CompilerParams(use_tc_tiling_on_sc=False,
                                             needs_layout_passes=False),
        scratch_types=[
            pltpu.VMEM((_NBUF, 1, _LANE), jnp.int32),
            pltpu.VMEM((_NBUF, _LANE, d), jnp.float32),
            pltpu.VMEM((_NBUF, blk_words), jnp.float32),
            pltpu.SemaphoreType.DMA((_NBUF,)),
            pltpu.SemaphoreType.DMA((_NBUF,)),
            pltpu.SemaphoreType.DMA((_NBUF,)),
        ],
    )
    def gather(idx_hbm, table_hbm, out_hbm, idx_v, rows_v, trans_v,
               sem_i, sem_g, sem_s):
        wid = lax.axis_index("s") * 2 + lax.axis_index("c")
        base = wid * rows_per_w

        iota = lax.iota(jnp.int32, 16)

        def fire_gather(j):
            pltpu.async_copy(table_hbm.at[idx_v.at[j, 0]], rows_v.at[j],
                             sem_g.at[j])

        def drain_gather(j):
            pltpu.make_async_copy(
                table_hbm.at[pl.ds(0, _LANE)], rows_v.at[j],
                sem_g.at[j]).wait()

        def fire_idx(j, k):
            pltpu.async_copy(idx_hbm.at[pl.ds(k, 1)], idx_v.at[j],
                             sem_i.at[j])

        def drain_idx(j):
            pltpu.make_async_copy(
                idx_hbm.at[pl.ds(0, 1)], idx_v.at[j], sem_i.at[j]).wait()

        def fire_store(j, k):
            t = k // _LANE
            c = lax.rem(k, _LANE)
            for r in range(d_tiles):
                off = (t * d_tiles * _LANE + r * _LANE + c) * 8 * _LANE
                pltpu.async_copy(
                    trans_v.at[j, pl.ds(r * 8 * _LANE, 8 * _LANE)],
                    out_hbm.at[pl.ds(off, 8 * _LANE)], sem_s.at[j])

        def drain_store(j):
            pltpu.make_async_copy(
                trans_v.at[j], out_hbm.at[pl.ds(0, blk_words)],
                sem_s.at[j]).wait()

        def transpose_block(j):
            @plsc.parallel_loop(0, 16, 1, unroll=2)
            def _t(e):
                rot = lax.bitwise_and(iota + e, 15)
                sbase = rot * _LANE + iota
                for db in range(d // 16):
                    dcol = rot + db * 16
                    for lb in range(_LANE // 16):
                        lrow = iota + lb * 16
                        vals = plsc.load_gather(rows_v.at[j], [lrow, dcol])
                        plsc.store_scatter(
                            trans_v.at[j],
                            [sbase + (db * 16 * _LANE + lb * 16)], vals)

        for j in range(_NBUF):
            pltpu.sync_copy(idx_hbm.at[pl.ds(base + j, 1)], idx_v.at[j])
            fire_gather(j)

        def body(r, carry):
            for j in range(_NBUF):
                k = base + r * _NBUF + j
                drain_gather(j)

                @pl.when(r < n_iters - 1)
                def _prefetch_idx():
                    fire_idx(j, k + _NBUF)

                @pl.when(r > 0)
                def _wait_prev_store():
                    drain_store(j)

                transpose_block(j)
                fire_store(j, k)

                @pl.when(r < n_iters - 1)
                def _refill():
                    drain_idx(j)
                    fire_gather(j)
            return carry

        lax.fori_loop(0, n_iters, body, 0, unroll=False)

        for j in range(_NBUF):
            drain_store(j)

    return gather


def kernel(x, emb):
    b0, b1 = x.shape
    v, d = emb.shape
    n = b0 * b1
    n_rows = n // _LANE
    idx = x.T.reshape(n_rows, _LANE).astype(jnp.int32)
    out = _build_gather(n_rows, d, v, b1)(idx, emb)
    out = out.reshape(b1, d // 8, _LANE, 8, _LANE)
    return out.transpose(2, 4, 0, 1, 3).reshape(b0, b1, d)
